# baseline (speedup 1.0000x reference)
import functools

import jax
import jax.numpy as jnp
from jax import lax
from jax.experimental import pallas as pl
from jax.experimental.pallas import tpu as pltpu
from jax.experimental.pallas import tpu_sc as plsc

N = 10000
E = 320000
D_NODE = 128
D_EDGE = 16

NC = 2
NS = 16
NW = NC * NS

BLK = 128
NB = E // BLK
NBT = NB // NW
NTAIL = NB - NBT * NW
CB = 13
NSUP = NBT // CB
CHW = CB * 8 * BLK
IDXN = NBT * BLK

ZROWS = 624
TAIL = N - NS * ZROWS

_mesh = plsc.VectorSubcoreMesh(core_axis_name="c", subcore_axis_name="s")


@functools.partial(
    pl.kernel,
    out_type=jax.ShapeDtypeStruct((NC, N, D_EDGE), jnp.float32),
    mesh=_mesh,
    scratch_types=[
        pltpu.VMEM((IDXN + BLK,), jnp.int32),
        pltpu.VMEM((IDXN + BLK,), jnp.int32),
        pltpu.VMEM((2 * CHW,), jnp.float32),
        pltpu.VMEM((2 * CHW,), jnp.float32),
        pltpu.VMEM((CB * BLK, D_EDGE), jnp.float32),
        pltpu.VMEM_SHARED((N, D_EDGE), jnp.float32),
        pltpu.SemaphoreType.DMA,
        pltpu.SemaphoreType.DMA,
    ],
    compiler_params=pltpu.CompilerParams(use_tc_tiling_on_sc=False,
                                         needs_layout_passes=False),
)
def _scatter_add_sc(eidx_hbm, z_hbm, zeros_hbm, out_hbm,
                    idx_s, idx_d, chunk_a, chunk_b, rowbuf, acc,
                    zsem_a, zsem_b):
    c = lax.axis_index("c")
    s = lax.axis_index("s")
    t = s * NC + c
    estart = t * IDXN
    pltpu.sync_copy(zeros_hbm, acc.at[pl.ds(s * ZROWS, ZROWS)])

    @pl.when(s == 0)
    def _zero_tail():
        pltpu.sync_copy(zeros_hbm.at[pl.ds(0, TAIL)],
                        acc.at[pl.ds(NS * ZROWS, TAIL)])

    pltpu.sync_copy(eidx_hbm.at[0, pl.ds(estart, IDXN)],
                    idx_s.at[pl.ds(0, IDXN)])
    pltpu.sync_copy(eidx_hbm.at[1, pl.ds(estart, IDXN)],
                    idx_d.at[pl.ds(0, IDXN)])

    @pl.when(t < NTAIL)
    def _tail_idx():
        tstart = NW * IDXN + t * BLK
        pltpu.sync_copy(eidx_hbm.at[0, pl.ds(tstart, BLK)],
                        idx_s.at[pl.ds(IDXN, BLK)])
        pltpu.sync_copy(eidx_hbm.at[1, pl.ds(tstart, BLK)],
                        idx_d.at[pl.ds(IDXN, BLK)])

    plsc.subcore_barrier()
    iota16 = lax.iota(jnp.int32, 16)

    def transpose_block(chunk, j):
        rows = [iota16 + (j * BLK + 16 * lg) for lg in range(8)]
        for k in range(D_EDGE):
            tr, rr = divmod(k, 8)
            col = jnp.full((16,), k, jnp.int32)
            off0 = tr * CHW + rr * BLK
            for lg in range(8):
                v = chunk[pl.ds(off0 + j * (8 * BLK) + 16 * lg, 16)]
                plsc.store_scatter(rowbuf, [rows[lg], col], v)

    def zdma_issue(u, chunk, zsem):
        zoff = (t * NBT + u * CB) * (8 * BLK)
        pltpu.async_copy(z_hbm.at[0, pl.ds(zoff, CHW)],
                         chunk.at[pl.ds(0, CHW)], zsem)
        pltpu.async_copy(z_hbm.at[1, pl.ds(zoff, CHW)],
                         chunk.at[pl.ds(CHW, CHW)], zsem)

    def zdma_wait(chunk, zsem):
        pltpu.make_async_copy(z_hbm.at[0, pl.ds(0, 2 * CHW)], chunk,
                              zsem).wait()

    def process(chunk, u):
        @plsc.parallel_loop(0, CB, 1, unroll=4)
        def _transpose_loop(j):
            transpose_block(chunk, j)

        ioff = u * CB * BLK
        pltpu.sync_copy(rowbuf, acc.at[idx_s.at[pl.ds(ioff, CB * BLK)]],
                        add=True)
        pltpu.sync_copy(rowbuf, acc.at[idx_d.at[pl.ds(ioff, CB * BLK)]],
                        add=True)

    zdma_issue(0, chunk_a, zsem_a)

    def pair_body(p, _):
        u0 = 2 * p
        zdma_issue(u0 + 1, chunk_b, zsem_b)
        zdma_wait(chunk_a, zsem_a)
        process(chunk_a, u0)

        @pl.when(p < NSUP // 2 - 1)
        def _prefetch():
            zdma_issue(u0 + 2, chunk_a, zsem_a)

        zdma_wait(chunk_b, zsem_b)
        process(chunk_b, u0 + 1)
        return 0

    lax.fori_loop(0, NSUP // 2, pair_body, 0)

    @pl.when(t < NTAIL)
    def _tail_block():
        zoff = (NW * NBT + t) * (8 * BLK)
        pltpu.sync_copy(z_hbm.at[0, pl.ds(zoff, 8 * BLK)],
                        chunk_a.at[pl.ds(0, 8 * BLK)])
        pltpu.sync_copy(z_hbm.at[1, pl.ds(zoff, 8 * BLK)],
                        chunk_a.at[pl.ds(CHW, 8 * BLK)])
        transpose_block(chunk_a, 0)
        rws = rowbuf.at[pl.ds(0, BLK)]
        pltpu.sync_copy(rws, acc.at[idx_s.at[pl.ds(IDXN, BLK)]], add=True)
        pltpu.sync_copy(rws, acc.at[idx_d.at[pl.ds(IDXN, BLK)]], add=True)

    plsc.subcore_barrier()
    pltpu.sync_copy(acc.at[pl.ds(s * ZROWS, ZROWS)],
                    out_hbm.at[c, pl.ds(s * ZROWS, ZROWS)])

    @pl.when(s == 0)
    def _flush_tail():
        pltpu.sync_copy(acc.at[pl.ds(NS * ZROWS, TAIL)],
                        out_hbm.at[c, pl.ds(NS * ZROWS, TAIL)])


def _linear_body(x_ref, p_ref, w1_ref, w2_ref, b_ref, o_ref):
    pb = p_ref[0] + p_ref[1]
    o_ref[...] = (
        jnp.dot(x_ref[...], w1_ref[...], preferred_element_type=jnp.float32)
        + jnp.dot(pb, w2_ref[...], preferred_element_type=jnp.float32)
        + b_ref[...]
    )


_BM = 2000


def _linear_tc(x_node, partials, W1, W2, b2d):
    return pl.pallas_call(
        _linear_body,
        grid=(N // _BM,),
        in_specs=[
            pl.BlockSpec((_BM, D_NODE), lambda i: (i, 0)),
            pl.BlockSpec((2, _BM, D_EDGE), lambda i: (0, i, 0)),
            pl.BlockSpec((D_NODE, D_NODE), lambda i: (0, 0)),
            pl.BlockSpec((D_EDGE, D_NODE), lambda i: (0, 0)),
            pl.BlockSpec((1, D_NODE), lambda i: (0, 0)),
        ],
        out_specs=pl.BlockSpec((_BM, D_NODE), lambda i: (i, 0)),
        out_shape=jax.ShapeDtypeStruct((N, D_NODE), jnp.float32),
    )(x_node, partials, W1, W2, b2d)


def kernel(x_node, x_edge, edge_index, W, b):
    z = (x_edge.T.reshape(2, 8, NB, BLK)
         .transpose(0, 2, 1, 3)
         .reshape(2, NB * 8 * BLK))
    zeros = jnp.zeros((ZROWS, D_EDGE), jnp.float32)
    partials = _scatter_add_sc(edge_index, z, zeros)
    x_node_out = _linear_tc(
        x_node, partials,
        W[:D_NODE], W[D_NODE:], b.reshape(1, D_NODE),
    )
    return (x_node_out, x_edge, edge_index)


# speedup vs baseline: 1.1626x; 1.1626x over previous
import functools

import jax
import jax.numpy as jnp
from jax import lax
from jax.experimental import pallas as pl
from jax.experimental.pallas import tpu as pltpu
from jax.experimental.pallas import tpu_sc as plsc

N = 10000
E = 320000
D_NODE = 128
D_EDGE = 16

NC = 2
NS = 16
NW = NC * NS

BLK = 128
NB = E // BLK
NBT = NB // NW
NTAIL = NB - NBT * NW
CB = 13
NSUP = NBT // CB
CHW = CB * 8 * BLK
IDXN = NBT * BLK

ZROWS = 624
TAIL = N - NS * ZROWS

_mesh = plsc.VectorSubcoreMesh(core_axis_name="c", subcore_axis_name="s")


@functools.partial(
    pl.kernel,
    out_type=jax.ShapeDtypeStruct((NC, N, D_EDGE), jnp.float32),
    mesh=_mesh,
    scratch_types=[
        pltpu.VMEM((IDXN + BLK,), jnp.int32),
        pltpu.VMEM((IDXN + BLK,), jnp.int32),
        pltpu.VMEM((2 * CHW,), jnp.float32),
        pltpu.VMEM((2 * CHW,), jnp.float32),
        pltpu.VMEM((CB * BLK, D_EDGE), jnp.float32),
        pltpu.VMEM_SHARED((N, D_EDGE), jnp.float32),
        pltpu.SemaphoreType.DMA,
        pltpu.SemaphoreType.DMA,
    ],
    compiler_params=pltpu.CompilerParams(use_tc_tiling_on_sc=False,
                                         needs_layout_passes=False),
)
def _scatter_add_sc(eidx_hbm, z_hbm, zeros_hbm, out_hbm,
                    idx_s, idx_d, chunk_a, chunk_b, rowbuf, acc,
                    zsem_a, zsem_b):
    c = lax.axis_index("c")
    s = lax.axis_index("s")
    t = s * NC + c
    estart = t * IDXN
    pltpu.sync_copy(zeros_hbm, acc.at[pl.ds(s * ZROWS, ZROWS)])

    @pl.when(s == 0)
    def _zero_tail():
        pltpu.sync_copy(zeros_hbm.at[pl.ds(0, TAIL)],
                        acc.at[pl.ds(NS * ZROWS, TAIL)])

    pltpu.sync_copy(eidx_hbm.at[0, pl.ds(estart, IDXN)],
                    idx_s.at[pl.ds(0, IDXN)])
    pltpu.sync_copy(eidx_hbm.at[1, pl.ds(estart, IDXN)],
                    idx_d.at[pl.ds(0, IDXN)])

    @pl.when(t < NTAIL)
    def _tail_idx():
        tstart = NW * IDXN + t * BLK
        pltpu.sync_copy(eidx_hbm.at[0, pl.ds(tstart, BLK)],
                        idx_s.at[pl.ds(IDXN, BLK)])
        pltpu.sync_copy(eidx_hbm.at[1, pl.ds(tstart, BLK)],
                        idx_d.at[pl.ds(IDXN, BLK)])

    plsc.subcore_barrier()
    iota16 = lax.iota(jnp.int32, 16)

    def transpose_block(chunk, j):
        rows = [iota16 + (j * BLK + 16 * lg) for lg in range(8)]
        for k in range(D_EDGE):
            tr, rr = divmod(k, 8)
            col = jnp.full((16,), k, jnp.int32)
            off0 = tr * CHW + rr * BLK
            for lg in range(8):
                v = chunk[pl.ds(off0 + j * (8 * BLK) + 16 * lg, 16)]
                plsc.store_scatter(rowbuf, [rows[lg], col], v)

    def zdma_issue(u, chunk, zsem):
        zoff = (t * NBT + u * CB) * (8 * BLK)
        pltpu.async_copy(z_hbm.at[0, pl.ds(zoff, CHW)],
                         chunk.at[pl.ds(0, CHW)], zsem)
        pltpu.async_copy(z_hbm.at[1, pl.ds(zoff, CHW)],
                         chunk.at[pl.ds(CHW, CHW)], zsem)

    def zdma_wait(chunk, zsem):
        pltpu.make_async_copy(z_hbm.at[0, pl.ds(0, 2 * CHW)], chunk,
                              zsem).wait()

    def process(chunk, u):
        @plsc.parallel_loop(0, CB, 1, unroll=2)
        def _transpose_loop(j):
            transpose_block(chunk, j)

        ioff = u * CB * BLK
        pltpu.sync_copy(rowbuf, acc.at[idx_s.at[pl.ds(ioff, CB * BLK)]],
                        add=True)
        pltpu.sync_copy(rowbuf, acc.at[idx_d.at[pl.ds(ioff, CB * BLK)]],
                        add=True)

    zdma_issue(0, chunk_a, zsem_a)

    def pair_body(p, _):
        u0 = 2 * p
        zdma_issue(u0 + 1, chunk_b, zsem_b)
        zdma_wait(chunk_a, zsem_a)
        process(chunk_a, u0)

        @pl.when(p < NSUP // 2 - 1)
        def _prefetch():
            zdma_issue(u0 + 2, chunk_a, zsem_a)

        zdma_wait(chunk_b, zsem_b)
        process(chunk_b, u0 + 1)
        return 0

    lax.fori_loop(0, NSUP // 2, pair_body, 0)

    @pl.when(t < NTAIL)
    def _tail_block():
        zoff = (NW * NBT + t) * (8 * BLK)
        pltpu.sync_copy(z_hbm.at[0, pl.ds(zoff, 8 * BLK)],
                        chunk_a.at[pl.ds(0, 8 * BLK)])
        pltpu.sync_copy(z_hbm.at[1, pl.ds(zoff, 8 * BLK)],
                        chunk_a.at[pl.ds(CHW, 8 * BLK)])
        transpose_block(chunk_a, 0)
        rws = rowbuf.at[pl.ds(0, BLK)]
        pltpu.sync_copy(rws, acc.at[idx_s.at[pl.ds(IDXN, BLK)]], add=True)
        pltpu.sync_copy(rws, acc.at[idx_d.at[pl.ds(IDXN, BLK)]], add=True)

    plsc.subcore_barrier()
    pltpu.sync_copy(acc.at[pl.ds(s * ZROWS, ZROWS)],
                    out_hbm.at[c, pl.ds(s * ZROWS, ZROWS)])

    @pl.when(s == 0)
    def _flush_tail():
        pltpu.sync_copy(acc.at[pl.ds(NS * ZROWS, TAIL)],
                        out_hbm.at[c, pl.ds(NS * ZROWS, TAIL)])


def _linear_body(x_ref, p_ref, w1_ref, w2_ref, b_ref, o_ref):
    pb = p_ref[0] + p_ref[1]
    o_ref[...] = (
        jnp.dot(x_ref[...], w1_ref[...], preferred_element_type=jnp.float32)
        + jnp.dot(pb, w2_ref[...], preferred_element_type=jnp.float32)
        + b_ref[...]
    )


_BM = 2000


def _linear_tc(x_node, partials, W1, W2, b2d):
    return pl.pallas_call(
        _linear_body,
        grid=(N // _BM,),
        in_specs=[
            pl.BlockSpec((_BM, D_NODE), lambda i: (i, 0)),
            pl.BlockSpec((2, _BM, D_EDGE), lambda i: (0, i, 0)),
            pl.BlockSpec((D_NODE, D_NODE), lambda i: (0, 0)),
            pl.BlockSpec((D_EDGE, D_NODE), lambda i: (0, 0)),
            pl.BlockSpec((1, D_NODE), lambda i: (0, 0)),
        ],
        out_specs=pl.BlockSpec((_BM, D_NODE), lambda i: (i, 0)),
        out_shape=jax.ShapeDtypeStruct((N, D_NODE), jnp.float32),
    )(x_node, partials, W1, W2, b2d)


def kernel(x_node, x_edge, edge_index, W, b):
    z = (x_edge.T.reshape(2, 8, NB, BLK)
         .transpose(0, 2, 1, 3)
         .reshape(2, NB * 8 * BLK))
    zeros = jnp.zeros((ZROWS, D_EDGE), jnp.float32)
    partials = _scatter_add_sc(edge_index, z, zeros)
    x_node_out = _linear_tc(
        x_node, partials,
        W[:D_NODE], W[D_NODE:], b.reshape(1, D_NODE),
    )
    return (x_node_out, x_edge, edge_index)
